# baseline (speedup 1.0000x reference)
import functools

import jax
import jax.numpy as jnp
from jax import lax
from jax.experimental import pallas as pl
from jax.experimental.pallas import tpu as pltpu
from jax.experimental.pallas import tpu_sc as plsc

NC = 2
NS = 16
NW = NC * NS

B_ALL = 16384
F = 26
D = 32
BBLK = 128
NBLK = B_ALL // BBLK
BLK_PER_W = NBLK // NW
CHUNK = BBLK * F


def _gather_tiled_out(v):
    mesh = plsc.VectorSubcoreMesh(
        core_axis_name="c", subcore_axis_name="s",
        num_cores=NC, num_subcores=NS)

    @functools.partial(
        pl.kernel,
        out_type=jax.ShapeDtypeStruct((B_ALL * F * D,), jnp.float32),
        mesh=mesh,
        compiler_params=pltpu.CompilerParams(
            use_tc_tiling_on_sc=False, needs_layout_passes=False),
        scratch_types=[
            pltpu.VMEM((CHUNK,), jnp.int32),
            pltpu.VMEM((BBLK,), jnp.int32),
            pltpu.VMEM((BBLK,), jnp.int32),
            pltpu.VMEM((BBLK, D), jnp.float32),
            pltpu.VMEM((BBLK, D), jnp.float32),
            pltpu.VMEM((4 * 8 * BBLK,), jnp.float32),
            pltpu.VMEM((4 * 8 * BBLK,), jnp.float32),
            pltpu.SemaphoreType.DMA,
            pltpu.SemaphoreType.DMA((2,)),
            pltpu.SemaphoreType.DMA((2,)),
        ],
    )
    def body(idx_hbm, tab_hbm, out_hbm, chunk_v, ib0, ib1, r0, r1, s0, s1,
             sem_c, sem_g, sem_o):
        ibuf = (ib0, ib1)
        rows = (r0, r1)
        stage = (s0, s1)
        wid = lax.axis_index("s") * NC + lax.axis_index("c")
        lane = lax.broadcasted_iota(jnp.int32, (16,), 0)

        def build_ibuf(s, f):
            for h in range(8):
                ids = (lane + 16 * h) * F + f
                v16 = plsc.load_gather(chunk_v, [ids])
                ibuf[s][pl.ds(16 * h, 16)] = v16

        def start_gather(s):
            return pltpu.async_copy(tab_hbm.at[ibuf[s]], rows[s],
                                    sem_g.at[s])

        def transpose_unit(s):
            def tr(i, _):
                for h in range(8):
                    b_ids = lane + 16 * h
                    e_ids = jnp.full((16,), 0, jnp.int32) + i
                    v16 = plsc.load_gather(rows[s], [b_ids, e_ids])
                    stage[s][pl.ds(i * BBLK + 16 * h, 16)] = v16
                return 0
            lax.fori_loop(0, D, tr, 0)

        def out_copies(s, f, blk, issue):
            waits = []
            for t in range(4):
                cp = pltpu.make_async_copy(
                    stage[s].at[pl.ds(t * 8 * BBLK, 8 * BBLK)],
                    out_hbm.at[pl.ds(((f * 4 + t) * NBLK + blk)
                                     * 8 * BBLK, 8 * BBLK)],
                    sem_o.at[s])
                if issue:
                    cp.start()
                else:
                    waits.append(cp)
            for cp in waits:
                cp.wait()

        def unit(s, f, blk, q):
            pltpu.make_async_copy(tab_hbm.at[ibuf[s]], rows[s],
                                  sem_g.at[s]).wait()
            nf = f + 2

            @pl.when(nf < F)
            def _():
                build_ibuf(s, nf)

            @pl.when(q > 0)
            def _():
                out_copies(s, f - 2, blk, issue=False)
            transpose_unit(s)
            out_copies(s, f, blk, issue=True)

            @pl.when(nf < F)
            def _():
                start_gather(s)

        def blk_body(m, _):
            blk = wid * BLK_PER_W + m
            pltpu.async_copy(
                idx_hbm.at[pl.ds(blk * CHUNK, CHUNK)], chunk_v, sem_c).wait()
            build_ibuf(0, 0)
            start_gather(0)
            build_ibuf(1, 1)
            start_gather(1)

            def q_body(q, _):
                unit(0, 2 * q, blk, q)
                unit(1, 2 * q + 1, blk, q)
                return 0

            lax.fori_loop(0, F // 2, q_body, 0)
            out_copies(0, F - 2, blk, issue=False)
            out_copies(1, F - 1, blk, issue=False)
            return 0

        lax.fori_loop(0, BLK_PER_W, blk_body, 0)

    return body(*v)


def kernel(inputs, embedding):
    b, f = inputs.shape
    vv, d = embedding.shape
    idx = inputs.reshape(-1).astype(jnp.int32)
    flat = _gather_tiled_out((idx, embedding))
    out = (flat.reshape(F, 4, NBLK, 8, BBLK)
           .transpose(2, 4, 0, 1, 3)
           .reshape(B_ALL, F, D))
    return out
